# baseline (speedup 1.0000x reference)
import functools

import jax
import jax.numpy as jnp
from jax.experimental import pallas as pl
from jax.experimental.pallas import tpu as pltpu

H = 256
CUTOFF = 10.0


def _f_update_body(f_ref, dih_ref, we_ref, be_ref, cwd_ref, bd_ref,
                   fout_ref, dinfo_ref):
    f = f_ref[...]
    d = dih_ref[...]
    dinfo = jnp.broadcast_to(d, (d.shape[0], H))
    dinfo_ref[...] = dinfo
    mod = jax.nn.sigmoid(d * cwd_ref[...] + bd_ref[...])
    f_lin = jnp.dot(f, we_ref[...], preferred_element_type=jnp.float32)
    fout_ref[...] = f + (f_lin + be_ref[...]) * mod


def _f_update(f, dihedral, lin_edge_w, lin_edge_b, lin_dihedral_w,
              lin_dihedral_b):
    E = f.shape[0]
    blk = 2000
    grid = E // blk
    cwd = jnp.sum(lin_dihedral_w, axis=0)[None, :]
    be = lin_edge_b[None, :]
    bd = lin_dihedral_b[None, :]
    dih = dihedral[:, None]
    fout, dinfo = pl.pallas_call(
        _f_update_body,
        grid=(grid,),
        in_specs=[
            pl.BlockSpec((blk, H), lambda i: (i, 0)),
            pl.BlockSpec((blk, 1), lambda i: (i, 0)),
            pl.BlockSpec((H, H), lambda i: (0, 0)),
            pl.BlockSpec((1, H), lambda i: (0, 0)),
            pl.BlockSpec((1, H), lambda i: (0, 0)),
            pl.BlockSpec((1, H), lambda i: (0, 0)),
        ],
        out_specs=[
            pl.BlockSpec((blk, H), lambda i: (i, 0)),
            pl.BlockSpec((blk, H), lambda i: (i, 0)),
        ],
        out_shape=[
            jax.ShapeDtypeStruct((E, H), jnp.float32),
            jax.ShapeDtypeStruct((E, H), jnp.float32),
        ],
    )(f, dih, lin_edge_w, be, cwd, bd)
    return fout, dinfo


def _node_update_body(h_ref, wab_ref, ws_ref, bs_ref, ang_ref, cwa_ref,
                      ba_ref, ab_ref, hout_ref, ainfo_ref):
    h = h_ref[...]
    ab_ref[...] = jnp.dot(h, wab_ref[...], preferred_element_type=jnp.float32)
    a = ang_ref[...]
    ainfo_ref[...] = jnp.broadcast_to(a, (a.shape[0], H))
    mod = jax.nn.sigmoid(a * cwa_ref[...] + ba_ref[...])
    h_lin = jnp.dot(h, ws_ref[...], preferred_element_type=jnp.float32)
    hout_ref[...] = h + (h_lin + bs_ref[...]) * mod


def _node_update(h, wab, lin_scalar_w, lin_scalar_b, angular, lin_angular_w,
                 lin_angular_b):
    N = h.shape[0]
    blk = 2000
    grid = N // blk
    cwa = jnp.sum(lin_angular_w, axis=0)[None, :]
    bs = lin_scalar_b[None, :]
    ba = lin_angular_b[None, :]
    ang = angular[:, None]
    ab, hout, ainfo = pl.pallas_call(
        _node_update_body,
        grid=(grid,),
        in_specs=[
            pl.BlockSpec((blk, H), lambda i: (i, 0)),
            pl.BlockSpec((H, 4 * H), lambda i: (0, 0)),
            pl.BlockSpec((H, H), lambda i: (0, 0)),
            pl.BlockSpec((1, H), lambda i: (0, 0)),
            pl.BlockSpec((blk, 1), lambda i: (i, 0)),
            pl.BlockSpec((1, H), lambda i: (0, 0)),
            pl.BlockSpec((1, H), lambda i: (0, 0)),
        ],
        out_specs=[
            pl.BlockSpec((blk, 4 * H), lambda i: (i, 0)),
            pl.BlockSpec((blk, H), lambda i: (i, 0)),
            pl.BlockSpec((blk, H), lambda i: (i, 0)),
        ],
        out_shape=[
            jax.ShapeDtypeStruct((N, 4 * H), jnp.float32),
            jax.ShapeDtypeStruct((N, H), jnp.float32),
            jax.ShapeDtypeStruct((N, H), jnp.float32),
        ],
    )(h, wab, lin_scalar_w, bs, ang, cwa, ba)
    return ab, hout, ainfo


def kernel(h, v, f, pos, edge_index, edge_rbf,
           lin_msg_w, lin_msg_b, lin_vec_w, lin_vec_b,
           lin_scalar_w, lin_scalar_b, lin_edge_w, lin_edge_b,
           lin_angular_w, lin_angular_b, lin_dihedral_w, lin_dihedral_b):
    row = edge_index[0]
    col = edge_index[1]
    n_nodes = pos.shape[0]

    edge_vec = pos[col] - pos[row]
    edge_dist = jnp.sqrt(jnp.sum(edge_vec * edge_vec, axis=-1,
                                 keepdims=True)) + 1e-8
    unit_vec = edge_vec / edge_dist

    du = jnp.zeros((n_nodes, 3), dtype=pos.dtype)
    du = du.at[row].add(unit_vec)
    du = du.at[col].add(-unit_vec)
    angular = jnp.sum(du * du, axis=1)

    v_i = du[row]
    v_j = du[col]
    dot_vi = jnp.sum(v_i * unit_vec, axis=-1, keepdims=True)
    dot_vj = jnp.sum(v_j * (-unit_vec), axis=-1, keepdims=True)
    w_ij = v_i - dot_vi * unit_vec
    w_ji = v_j - dot_vj * (-unit_vec)
    dihedral = jnp.sum(w_ij * w_ji, axis=-1)

    wf = lin_msg_w @ lin_vec_w
    bf = lin_msg_b @ lin_vec_w + lin_vec_b
    wab = jnp.concatenate([wf[:H], wf[H:2 * H]], axis=1)

    ab, h_updated, angular_info = _node_update(
        h, wab, lin_scalar_w, lin_scalar_b, angular,
        lin_angular_w, lin_angular_b)
    a_tab = ab[:, :2 * H]
    b_tab = ab[:, 2 * H:]

    crbf = edge_rbf @ wf[2 * H:] + bf
    vw = a_tab[col] + b_tab[row] + crbf
    d = edge_dist[:, 0]
    cw = 0.5 * (jnp.cos(jnp.pi * d / CUTOFF) + 1.0) * (d < CUTOFF)
    w1 = vw[:, :H] * cw[:, None]
    w2 = vw[:, H:] * cw[:, None]
    vec_msg = w1[:, None, :] * unit_vec[:, :, None] + w2[:, None, :] * v[row]
    v_updated = v + jnp.zeros_like(v).at[col].add(vec_msg)

    f_updated, dihedral_info = _f_update(
        f, dihedral, lin_edge_w, lin_edge_b, lin_dihedral_w, lin_dihedral_b)

    return (h_updated, v_updated, f_updated, angular_info, dihedral_info, du)


# speedup vs baseline: 3.0521x; 3.0521x over previous
import functools

import jax
import jax.numpy as jnp
from jax import lax
from jax.experimental import pallas as pl
from jax.experimental.pallas import tpu as pltpu
from jax.experimental.pallas import tpu_sc as plsc

H = 256
CUTOFF = 10.0

_NB = 32
_NW = 32


def _f_update_body(f_ref, dih_ref, we_ref, be_ref, cwd_ref, bd_ref,
                   fout_ref, dinfo_ref):
    f = f_ref[...]
    d = dih_ref[...]
    dinfo_ref[...] = jnp.broadcast_to(d, (d.shape[0], H))
    mod = jax.nn.sigmoid(d * cwd_ref[...] + bd_ref[...])
    f_lin = jnp.dot(f, we_ref[...], preferred_element_type=jnp.float32)
    fout_ref[...] = f + (f_lin + be_ref[...]) * mod


def _f_update(f, dihedral, lin_edge_w, lin_edge_b, lin_dihedral_w,
              lin_dihedral_b):
    E = f.shape[0]
    blk = 2000
    cwd = jnp.sum(lin_dihedral_w, axis=0)[None, :]
    fout, dinfo = pl.pallas_call(
        _f_update_body,
        grid=(E // blk,),
        in_specs=[
            pl.BlockSpec((blk, H), lambda i: (i, 0)),
            pl.BlockSpec((blk, 1), lambda i: (i, 0)),
            pl.BlockSpec((H, H), lambda i: (0, 0)),
            pl.BlockSpec((1, H), lambda i: (0, 0)),
            pl.BlockSpec((1, H), lambda i: (0, 0)),
            pl.BlockSpec((1, H), lambda i: (0, 0)),
        ],
        out_specs=[
            pl.BlockSpec((blk, H), lambda i: (i, 0)),
            pl.BlockSpec((blk, H), lambda i: (i, 0)),
        ],
        out_shape=[
            jax.ShapeDtypeStruct((E, H), jnp.float32),
            jax.ShapeDtypeStruct((E, H), jnp.float32),
        ],
    )(f, dihedral[:, None], lin_edge_w, lin_edge_b[None, :], cwd,
      lin_dihedral_b[None, :])
    return fout, dinfo


def _node_update_body(h_ref, wab_ref, ws_ref, bs_ref, ang_ref, cwa_ref,
                      ba_ref, ab_ref, hout_ref, ainfo_ref):
    h = h_ref[...]
    ab_ref[...] = jnp.dot(h, wab_ref[...], preferred_element_type=jnp.float32,
                          precision=lax.Precision.HIGHEST)
    a = ang_ref[...]
    ainfo_ref[...] = jnp.broadcast_to(a, (a.shape[0], H))
    mod = jax.nn.sigmoid(a * cwa_ref[...] + ba_ref[...])
    h_lin = jnp.dot(h, ws_ref[...], preferred_element_type=jnp.float32)
    hout_ref[...] = h + (h_lin + bs_ref[...]) * mod


def _node_update(h, wab, lin_scalar_w, lin_scalar_b, angular, lin_angular_w,
                 lin_angular_b):
    N = h.shape[0]
    blk = 2000
    cwa = jnp.sum(lin_angular_w, axis=0)[None, :]
    ab, hout, ainfo = pl.pallas_call(
        _node_update_body,
        grid=(N // blk,),
        in_specs=[
            pl.BlockSpec((blk, H), lambda i: (i, 0)),
            pl.BlockSpec((H, 4 * H), lambda i: (0, 0)),
            pl.BlockSpec((H, H), lambda i: (0, 0)),
            pl.BlockSpec((1, H), lambda i: (0, 0)),
            pl.BlockSpec((blk, 1), lambda i: (i, 0)),
            pl.BlockSpec((1, H), lambda i: (0, 0)),
            pl.BlockSpec((1, H), lambda i: (0, 0)),
        ],
        out_specs=[
            pl.BlockSpec((blk, 4 * H), lambda i: (i, 0)),
            pl.BlockSpec((blk, H), lambda i: (i, 0)),
            pl.BlockSpec((blk, H), lambda i: (i, 0)),
        ],
        out_shape=[
            jax.ShapeDtypeStruct((N, 4 * H), jnp.float32),
            jax.ShapeDtypeStruct((N, H), jnp.float32),
            jax.ShapeDtypeStruct((N, H), jnp.float32),
        ],
    )(h, wab, lin_scalar_w, lin_scalar_b[None, :], angular[:, None], cwa,
      lin_angular_b[None, :])
    return ab, hout, ainfo


def _vec_msg_body(N, NPAD, E, a0, a1, b0, b1, vt00, vt01, vt10, vt11, vt20, vt21,
                  cperm, cubx, cwbx, rowi, coli, vout,
                  accum, idxc, idxr, abuf, bbuf, cbuf, vbuf, msgbuf, zbuf,
                  cub, cwb, semA, semB, semV):
    cid = lax.axis_index("c")
    sid = lax.axis_index("s")
    wid = cid * 16 + sid
    rows_per_tile = NPAD // 16

    vtabs = ((vt00, vt01), (vt10, vt11), (vt20, vt21))
    ab_tabs = ((a0, b0), (a1, b1))

    def _z(i, _):
        for r in range(8):
            zbuf[i, pl.ds(r * 16, 16)] = jnp.zeros((16,), jnp.float32)
        return 0
    lax.fori_loop(0, zbuf.shape[0], _z, 0)

    n_glob_batches = E // _NB
    base_batches = n_glob_batches // _NW
    extra = n_glob_batches - base_batches * _NW
    my_batches = jnp.where(wid < extra, base_batches + 1, base_batches)

    for d in range(3):
        for jh in range(2):
            chunk = d * 2 + jh
            a_t, b_t = ab_tabs[jh]
            v_t = vtabs[d][jh]

            for z in range(rows_per_tile // zbuf.shape[0]):
                pltpu.sync_copy(
                    zbuf,
                    accum.at[pl.ds(sid * rows_per_tile + z * zbuf.shape[0],
                                   zbuf.shape[0]), :])
            plsc.subcore_barrier()

            def batch_body(b, _):
                e0 = (wid + b * _NW) * _NB
                pltpu.sync_copy(coli.at[pl.ds(e0, _NB)], idxc)
                pltpu.sync_copy(rowi.at[pl.ds(e0, _NB)], idxr)
                cpA = pltpu.async_copy(a_t.at[idxc], abuf, semA)
                cpB = pltpu.async_copy(b_t.at[idxr], bbuf, semB)
                cpV = pltpu.async_copy(v_t.at[idxr], vbuf, semV)
                pltpu.sync_copy(
                    cperm.at[pl.ds(e0, _NB), pl.ds(jh * 256, 256)], cbuf)
                pltpu.sync_copy(cubx.at[d, pl.ds(e0, _NB), :], cub)
                pltpu.sync_copy(cwbx.at[pl.ds(e0, _NB), :], cwb)
                cpA.wait()
                cpB.wait()
                cpV.wait()

                def edge(e, _):
                    cud = cub[e, :]
                    cwe = cwb[e, :]
                    for r in range(8):
                        s1 = pl.ds(r * 16, 16)
                        s2 = pl.ds(128 + r * 16, 16)
                        w1 = abuf[e, s1] + bbuf[e, s1] + cbuf[e, s1]
                        w2 = abuf[e, s2] + bbuf[e, s2] + cbuf[e, s2]
                        msgbuf[e, s1] = w1 * cud + w2 * (vbuf[e, s1] * cwe)
                    return 0
                lax.fori_loop(0, _NB, edge, 0)

                pltpu.sync_copy(msgbuf, accum.at[idxc], add=True)
                return 0
            lax.fori_loop(0, my_batches, batch_body, 0)
            plsc.subcore_barrier()

            pltpu.sync_copy(
                accum.at[pl.ds(sid * rows_per_tile, rows_per_tile), :],
                vout.at[pl.ds((chunk * 2 + cid) * NPAD + sid * rows_per_tile,
                              rows_per_tile), :])
            plsc.subcore_barrier()


def _vec_msg_scatter(N, E, a0, a1, b0, b1, vtabs6, cperm, cubx, cwbx,
                     rowi, coli):
    npad = ((N + 255) // 256) * 256
    mesh = plsc.VectorSubcoreMesh(core_axis_name="c", subcore_axis_name="s")
    body = functools.partial(_vec_msg_body, N, npad, E)
    kfn = pl.kernel(
        body,
        out_type=jax.ShapeDtypeStruct((12 * npad, 128), jnp.float32),
        mesh=mesh,
        scratch_types=[
            pltpu.VMEM_SHARED((npad, 128), jnp.float32),
            pltpu.VMEM((_NB,), jnp.int32),
            pltpu.VMEM((_NB,), jnp.int32),
            pltpu.VMEM((_NB, 256), jnp.float32),
            pltpu.VMEM((_NB, 256), jnp.float32),
            pltpu.VMEM((_NB, 256), jnp.float32),
            pltpu.VMEM((_NB, 128), jnp.float32),
            pltpu.VMEM((_NB, 128), jnp.float32),
            pltpu.VMEM((16, 128), jnp.float32),
            pltpu.VMEM((_NB, 16), jnp.float32),
            pltpu.VMEM((_NB, 16), jnp.float32),
            pltpu.SemaphoreType.DMA,
            pltpu.SemaphoreType.DMA,
            pltpu.SemaphoreType.DMA,
        ],
    )
    return kfn(a0, a1, b0, b1, *vtabs6, cperm, cubx, cwbx, rowi, coli)


def kernel(h, v, f, pos, edge_index, edge_rbf,
           lin_msg_w, lin_msg_b, lin_vec_w, lin_vec_b,
           lin_scalar_w, lin_scalar_b, lin_edge_w, lin_edge_b,
           lin_angular_w, lin_angular_b, lin_dihedral_w, lin_dihedral_b):
    row = edge_index[0]
    col = edge_index[1]
    n_nodes = pos.shape[0]
    n_edges = row.shape[0]

    edge_vec = pos[col] - pos[row]
    edge_dist = jnp.sqrt(jnp.sum(edge_vec * edge_vec, axis=-1,
                                 keepdims=True)) + 1e-8
    unit_vec = edge_vec / edge_dist

    du = jnp.zeros((n_nodes, 3), dtype=pos.dtype)
    du = du.at[row].add(unit_vec)
    du = du.at[col].add(-unit_vec)
    angular = jnp.sum(du * du, axis=1)

    v_i = du[row]
    v_j = du[col]
    dot_vi = jnp.sum(v_i * unit_vec, axis=-1, keepdims=True)
    dot_vj = jnp.sum(v_j * (-unit_vec), axis=-1, keepdims=True)
    w_ij = v_i - dot_vi * unit_vec
    w_ji = v_j - dot_vj * (-unit_vec)
    dihedral = jnp.sum(w_ij * w_ji, axis=-1)

    wf = jnp.dot(lin_msg_w, lin_vec_w, precision=lax.Precision.HIGHEST)
    bf = lin_msg_b @ lin_vec_w + lin_vec_b
    perm = jnp.concatenate([
        jnp.arange(0, 128), jnp.arange(256, 384),
        jnp.arange(128, 256), jnp.arange(384, 512)])
    wfp = wf[:, perm]
    bfp = bf[perm]
    wab = jnp.concatenate([wfp[:H], wfp[H:2 * H]], axis=1)

    ab, h_updated, angular_info = _node_update(
        h, wab, lin_scalar_w, lin_scalar_b, angular,
        lin_angular_w, lin_angular_b)
    a0 = ab[:, 0:256]
    a1 = ab[:, 256:512]
    b0 = ab[:, 512:768]
    b1 = ab[:, 768:1024]

    cperm = jnp.dot(edge_rbf, wfp[2 * H:],
                    precision=lax.Precision.HIGHEST) + bfp

    d = edge_dist[:, 0]
    cw = 0.5 * (jnp.cos(jnp.pi * d / CUTOFF) + 1.0) * (d < CUTOFF)
    cu = cw[:, None] * unit_vec
    cubx = jnp.broadcast_to(cu.T[:, :, None], (3, n_edges, 16))
    cwbx = jnp.broadcast_to(cw[:, None], (n_edges, 16))

    vtabs6 = [v[:, dd, jj * 128:(jj + 1) * 128]
              for dd in range(3) for jj in range(2)]

    vout = _vec_msg_scatter(n_nodes, n_edges, a0, a1, b0, b1, vtabs6,
                            cperm, cubx, cwbx, row, col)
    npad = ((n_nodes + 255) // 256) * 256
    acc = vout.reshape(3, 2, 2, npad, 128)[:, :, :, :n_nodes, :].sum(axis=2)
    acc = acc.transpose(2, 0, 1, 3).reshape(n_nodes, 3, 2 * H // 2)
    v_updated = v + acc

    f_updated, dihedral_info = _f_update(
        f, dihedral, lin_edge_w, lin_edge_b, lin_dihedral_w, lin_dihedral_b)

    return (h_updated, v_updated, f_updated, angular_info, dihedral_info, du)
